# Initial kernel scaffold; baseline (speedup 1.0000x reference)
import jax
import jax.numpy as jnp
from jax.experimental import pallas as pl


def kernel(TRA_v_gene, TRA_j_gene, W_v, W_j):
    raise NotImplementedError("write your pallas kernel here")


# speedup vs baseline: 3.1244x; 3.1244x over previous
import functools

import jax
import jax.numpy as jnp
from jax import lax
from jax.experimental import pallas as pl
from jax.experimental.pallas import tpu as pltpu
from jax.experimental.pallas import tpu_sc as plsc

B = 16384
V = 1000
D_V = 16
D_J = 8
D_O = D_V + D_J

_INFO = plsc.get_sparse_core_info()
_NC, _NS, _L = _INFO.num_cores, _INFO.num_subcores, _INFO.num_lanes
_NW = _NC * _NS
_BPW = B // _NW
_PAIRS = _BPW // 2
_OUTW = _BPW * D_O


def _sc_body(idxv_hbm, idxj_hbm, wv_hbm, wj_hbm, out_hbm,
             idxv_v, idxj_v, wv_v, wj_v, comb_v):
    wid = lax.axis_index("s") * _NC + lax.axis_index("c")
    base = wid * _BPW

    pltpu.sync_copy(idxv_hbm.at[pl.ds(base, _BPW)], idxv_v)
    pltpu.sync_copy(idxj_hbm.at[pl.ds(base, _BPW)], idxj_v)
    pltpu.sync_copy(wv_hbm, wv_v)
    pltpu.sync_copy(wj_hbm, wj_v)

    iota = lax.iota(jnp.int32, _L)
    pos = [iota, _L + iota, 2 * _L + iota]
    roff = [jnp.where(p >= D_O, 1, 0) for p in pos]
    col = [p - roff_u * D_O for p, roff_u in zip(pos, roff)]
    vmask = [c < D_V for c in col]
    av_off = [jnp.where(m, c, 0) * 1 for c, m in zip(col, vmask)]
    aj_off = [jnp.where(m, 0, c - D_V) for c, m in zip(col, vmask)]

    def body(i, _):
        r0 = 2 * i
        out_base = i * (3 * _L)
        for u in range(3):
            r = r0 + roff[u]
            rv = plsc.load_gather(idxv_v, [r])
            addr_v = jnp.where(vmask[u], rv * D_V + av_off[u], 0)
            xv = plsc.load_gather(wv_v, [addr_v], mask=vmask[u])
            if u == 0:
                x = xv
            else:
                rj = plsc.load_gather(idxj_v, [r])
                addr_j = jnp.where(vmask[u], 0, rj * D_J + aj_off[u])
                xj = plsc.load_gather(wj_v, [addr_j], mask=~vmask[u])
                x = jnp.where(vmask[u], xv, xj)
            comb_v[pl.ds(out_base + u * _L, _L)] = x
        return _

    lax.fori_loop(0, _PAIRS, body, None)
    pltpu.sync_copy(comb_v, out_hbm.at[pl.ds(base * D_O, _OUTW)])


@jax.jit
def _gene_encode(idxv, idxj, wv_flat, wj_flat):
    mesh = plsc.VectorSubcoreMesh(core_axis_name="c", subcore_axis_name="s")
    k = functools.partial(
        pl.kernel,
        mesh=mesh,
        compiler_params=pltpu.CompilerParams(needs_layout_passes=False),
        out_type=jax.ShapeDtypeStruct((B * D_O,), jnp.float32),
        scratch_types=[
            pltpu.VMEM((_BPW,), jnp.int32),
            pltpu.VMEM((_BPW,), jnp.int32),
            pltpu.VMEM((V * D_V,), jnp.float32),
            pltpu.VMEM((V * D_J,), jnp.float32),
            pltpu.VMEM((_OUTW,), jnp.float32),
        ],
    )(_sc_body)
    return k(idxv, idxj, wv_flat, wj_flat)


def kernel(TRA_v_gene, TRA_j_gene, W_v, W_j):
    z_flat = _gene_encode(
        TRA_v_gene.astype(jnp.int32),
        TRA_j_gene.astype(jnp.int32),
        W_v.reshape(-1),
        W_j.reshape(-1),
    )
    return z_flat.reshape(B, D_O)


# speedup vs baseline: 3.7969x; 1.2153x over previous
import functools

import jax
import jax.numpy as jnp
from jax import lax
from jax.experimental import pallas as pl
from jax.experimental.pallas import tpu as pltpu
from jax.experimental.pallas import tpu_sc as plsc

B = 16384
V = 1000
D_V = 16
D_J = 8
D_O = D_V + D_J

_INFO = plsc.get_sparse_core_info()
_NC, _NS, _L = _INFO.num_cores, _INFO.num_subcores, _INFO.num_lanes
_NW = _NC * _NS
_BPW = B // _NW
_PAIRS = _BPW // 2
_OUTW = _BPW * D_O


def _sc_body(idxv_hbm, idxj_hbm, wv_hbm, wj_hbm, out_hbm,
             idxv_v, idxj_v, wv_v, wj_v, comb_v, sem):
    wid = lax.axis_index("s") * _NC + lax.axis_index("c")
    base = wid * _BPW

    cp1 = pltpu.async_copy(idxv_hbm.at[pl.ds(base, _BPW)], idxv_v, sem)
    cp2 = pltpu.async_copy(idxj_hbm.at[pl.ds(base, _BPW)], idxj_v, sem)
    cp3 = pltpu.async_copy(wv_hbm, wv_v, sem)
    cp4 = pltpu.async_copy(wj_hbm, wj_v, sem)
    cp1.wait()
    cp2.wait()
    cp3.wait()
    cp4.wait()

    iota = lax.iota(jnp.int32, _L)
    pos = [iota, _L + iota, 2 * _L + iota]
    roff = [jnp.where(p >= D_O, 1, 0) for p in pos]
    col = [p - roff_u * D_O for p, roff_u in zip(pos, roff)]
    vmask = [c < D_V for c in col]
    av_off = [jnp.where(m, c, 0) * 1 for c, m in zip(col, vmask)]
    aj_off = [jnp.where(m, 0, c - D_V) for c, m in zip(col, vmask)]

    @plsc.parallel_loop(0, _PAIRS, unroll=8)
    def _pair(i):
        r0 = 2 * i
        out_base = i * (3 * _L)
        for u in range(3):
            r = r0 + roff[u]
            rv = plsc.load_gather(idxv_v, [r])
            addr_v = jnp.where(vmask[u], rv * D_V + av_off[u], 0)
            xv = plsc.load_gather(wv_v, [addr_v], mask=vmask[u])
            if u == 0:
                x = xv
            else:
                rj = plsc.load_gather(idxj_v, [r])
                addr_j = jnp.where(vmask[u], 0, rj * D_J + aj_off[u])
                xj = plsc.load_gather(wj_v, [addr_j], mask=~vmask[u])
                x = jnp.where(vmask[u], xv, xj)
            comb_v[pl.ds(out_base + u * _L, _L)] = x
    pltpu.sync_copy(comb_v, out_hbm.at[pl.ds(base * D_O, _OUTW)])


@jax.jit
def _gene_encode(idxv, idxj, wv_flat, wj_flat):
    mesh = plsc.VectorSubcoreMesh(core_axis_name="c", subcore_axis_name="s")
    k = functools.partial(
        pl.kernel,
        mesh=mesh,
        compiler_params=pltpu.CompilerParams(needs_layout_passes=False),
        out_type=jax.ShapeDtypeStruct((B * D_O,), jnp.float32),
        scratch_types=[
            pltpu.VMEM((_BPW,), jnp.int32),
            pltpu.VMEM((_BPW,), jnp.int32),
            pltpu.VMEM((V * D_V,), jnp.float32),
            pltpu.VMEM((V * D_J,), jnp.float32),
            pltpu.VMEM((_OUTW,), jnp.float32),
            pltpu.SemaphoreType.DMA,
        ],
    )(_sc_body)
    return k(idxv, idxj, wv_flat, wj_flat)


def kernel(TRA_v_gene, TRA_j_gene, W_v, W_j):
    z_flat = _gene_encode(
        TRA_v_gene.astype(jnp.int32),
        TRA_j_gene.astype(jnp.int32),
        W_v.reshape(-1),
        W_j.reshape(-1),
    )
    return z_flat.reshape(B, D_O)


# speedup vs baseline: 4.3239x; 1.1388x over previous
import functools

import jax
import jax.numpy as jnp
from jax import lax
from jax.experimental import pallas as pl
from jax.experimental.pallas import tpu as pltpu
from jax.experimental.pallas import tpu_sc as plsc

B = 16384
V = 1000
D_V = 16
D_J = 8
D_O = D_V + D_J

_INFO = plsc.get_sparse_core_info()
_NC, _NS, _L = _INFO.num_cores, _INFO.num_subcores, _INFO.num_lanes
_NW = _NC * _NS
_BPW = B // _NW


def _sc_body(idxv_hbm, idxj_hbm, wv_hbm, wj_hbm, out_hbm,
             idxv_v, idxj_v, wv_v, wj_v, comb_v, sem):
    wid = lax.axis_index("s") * _NC + lax.axis_index("c")
    base = wid * _BPW

    cp1 = pltpu.async_copy(idxv_hbm.at[pl.ds(base, _BPW)], idxv_v, sem)
    cp2 = pltpu.async_copy(idxj_hbm.at[pl.ds(base, _BPW)], idxj_v, sem)
    cp3 = pltpu.async_copy(wv_hbm, wv_v, sem)
    cp4 = pltpu.async_copy(wj_hbm, wj_v, sem)
    cp1.wait()
    cp2.wait()
    cp3.wait()
    cp4.wait()

    iota = lax.iota(jnp.int32, _L)
    lo8 = iota < 8
    bv_off = 8 + iota
    bj_off = iota - 8

    @plsc.parallel_loop(0, _BPW, unroll=8)
    def _row(r):
        rvec = jnp.broadcast_to(r, (_L,)).astype(jnp.int32)
        rv = plsc.load_gather(idxv_v, [rvec])
        rj = plsc.load_gather(idxj_v, [rvec])
        x0 = plsc.load_gather(wv_v, [rv * D_V + iota])
        av = jnp.where(lo8, rv * D_V + bv_off, 0)
        aj = jnp.where(lo8, 0, rj * D_J + bj_off)
        xb = jnp.where(
            lo8,
            plsc.load_gather(wv_v, [av], mask=lo8),
            plsc.load_gather(wj_v, [aj], mask=~lo8),
        )
        comb_v[r, pl.ds(0, _L)] = x0
        comb_v[r, pl.ds(8, _L)] = xb

    pltpu.sync_copy(comb_v, out_hbm.at[pl.ds(base, _BPW)])


@jax.jit
def _gene_encode(idxv, idxj, wv_flat, wj_flat):
    mesh = plsc.VectorSubcoreMesh(core_axis_name="c", subcore_axis_name="s")
    k = functools.partial(
        pl.kernel,
        mesh=mesh,
        compiler_params=pltpu.CompilerParams(needs_layout_passes=False),
        out_type=jax.ShapeDtypeStruct((B, D_O), jnp.float32),
        scratch_types=[
            pltpu.VMEM((_BPW,), jnp.int32),
            pltpu.VMEM((_BPW,), jnp.int32),
            pltpu.VMEM((V * D_V,), jnp.float32),
            pltpu.VMEM((V * D_J,), jnp.float32),
            pltpu.VMEM((_BPW, D_O), jnp.float32),
            pltpu.SemaphoreType.DMA,
        ],
    )(_sc_body)
    return k(idxv, idxj, wv_flat, wj_flat)


def kernel(TRA_v_gene, TRA_j_gene, W_v, W_j):
    return _gene_encode(
        TRA_v_gene.astype(jnp.int32),
        TRA_j_gene.astype(jnp.int32),
        W_v.reshape(-1),
        W_j.reshape(-1),
    )


# speedup vs baseline: 5.5961x; 1.2942x over previous
import functools

import jax
import jax.numpy as jnp
from jax import lax
from jax.experimental import pallas as pl
from jax.experimental.pallas import tpu as pltpu
from jax.experimental.pallas import tpu_sc as plsc

B = 16384
V = 1000
D_V = 16
D_J = 8
D_O = D_V + D_J

_INFO = plsc.get_sparse_core_info()
_NC, _NS, _L = _INFO.num_cores, _INFO.num_subcores, _INFO.num_lanes
_NW = _NC * _NS
_BPW = B // _NW
_GRP = _BPW // _L


def _sc_body(idxv_hbm, idxj_hbm, wv_hbm, wj_hbm, out_hbm,
             idxv_v, idxj_v, wv_v, wj_v, comb_v, sem):
    wid = lax.axis_index("s") * _NC + lax.axis_index("c")
    base = wid * _BPW

    cp1 = pltpu.async_copy(idxv_hbm.at[pl.ds(base, _BPW)], idxv_v, sem)
    cp2 = pltpu.async_copy(idxj_hbm.at[pl.ds(base, _BPW)], idxj_v, sem)
    cp3 = pltpu.async_copy(wv_hbm, wv_v, sem)
    cp4 = pltpu.async_copy(wj_hbm, wj_v, sem)
    cp1.wait()
    cp2.wait()
    cp3.wait()
    cp4.wait()

    @plsc.parallel_loop(0, _GRP, unroll=4)
    def _group(g):
        off = g * _L
        ivv = idxv_v[pl.ds(off, _L)]
        ivj = idxj_v[pl.ds(off, _L)]
        av = ivv * D_V
        aj = ivj * D_J
        for c in range(D_V):
            comb_v[c, pl.ds(off, _L)] = plsc.load_gather(wv_v, [av + c])
        for c in range(D_J):
            comb_v[D_V + c, pl.ds(off, _L)] = plsc.load_gather(wj_v, [aj + c])

    pltpu.sync_copy(comb_v, out_hbm.at[:, pl.ds(base, _BPW)])


@jax.jit
def _gene_encode(idxv, idxj, wv_flat, wj_flat):
    mesh = plsc.VectorSubcoreMesh(core_axis_name="c", subcore_axis_name="s")
    k = functools.partial(
        pl.kernel,
        mesh=mesh,
        compiler_params=pltpu.CompilerParams(needs_layout_passes=False),
        out_type=jax.ShapeDtypeStruct((D_O, B), jnp.float32),
        scratch_types=[
            pltpu.VMEM((_BPW,), jnp.int32),
            pltpu.VMEM((_BPW,), jnp.int32),
            pltpu.VMEM((V * D_V,), jnp.float32),
            pltpu.VMEM((V * D_J,), jnp.float32),
            pltpu.VMEM((D_O, _BPW), jnp.float32),
            pltpu.SemaphoreType.DMA,
        ],
    )(_sc_body)
    return k(idxv, idxj, wv_flat, wj_flat)


def kernel(TRA_v_gene, TRA_j_gene, W_v, W_j):
    zt = _gene_encode(
        TRA_v_gene.astype(jnp.int32),
        TRA_j_gene.astype(jnp.int32),
        W_v.reshape(-1),
        W_j.reshape(-1),
    )
    return zt.T


# speedup vs baseline: 5.8835x; 1.0514x over previous
import functools

import jax
import jax.numpy as jnp
from jax import lax
from jax.experimental import pallas as pl
from jax.experimental.pallas import tpu as pltpu
from jax.experimental.pallas import tpu_sc as plsc

B = 16384
V = 1000
D_V = 16
D_J = 8
D_O = D_V + D_J

_INFO = plsc.get_sparse_core_info()
_NC, _NS, _L = _INFO.num_cores, _INFO.num_subcores, _INFO.num_lanes
_NW = _NC * _NS
_BPW = B // _NW
_GRP = _BPW // _L


def _sc_body(idxv_hbm, idxj_hbm, wvt_hbm, wjt_hbm, out_hbm,
             idxv_v, idxj_v, wv_v, wj_v, comb_v, sem):
    wid = lax.axis_index("s") * _NC + lax.axis_index("c")
    base = wid * _BPW

    cps = [pltpu.async_copy(idxv_hbm.at[pl.ds(base, _BPW)], idxv_v, sem),
           pltpu.async_copy(idxj_hbm.at[pl.ds(base, _BPW)], idxj_v, sem),
           pltpu.async_copy(wvt_hbm, wv_v, sem),
           pltpu.async_copy(wjt_hbm, wj_v, sem)]
    for cp in cps:
        cp.wait()

    cvecs = [jnp.broadcast_to(jnp.int32(c), (_L,)) for c in range(D_V)]

    @plsc.parallel_loop(0, _GRP, unroll=4)
    def _group(g):
        off = g * _L
        ivv = idxv_v[pl.ds(off, _L)]
        ivj = idxj_v[pl.ds(off, _L)]
        for c in range(D_V):
            comb_v[c, pl.ds(off, _L)] = plsc.load_gather(wv_v, [cvecs[c], ivv])
        for c in range(D_J):
            comb_v[D_V + c, pl.ds(off, _L)] = plsc.load_gather(
                wj_v, [cvecs[c], ivj])

    pltpu.sync_copy(comb_v, out_hbm.at[:, pl.ds(base, _BPW)])


@jax.jit
def _gene_encode(idxv, idxj, wvt, wjt):
    mesh = plsc.VectorSubcoreMesh(core_axis_name="c", subcore_axis_name="s")
    k = functools.partial(
        pl.kernel,
        mesh=mesh,
        compiler_params=pltpu.CompilerParams(needs_layout_passes=False),
        out_type=jax.ShapeDtypeStruct((D_O, B), jnp.float32),
        scratch_types=[
            pltpu.VMEM((_BPW,), jnp.int32),
            pltpu.VMEM((_BPW,), jnp.int32),
            pltpu.VMEM((D_V, V), jnp.float32),
            pltpu.VMEM((D_J, V), jnp.float32),
            pltpu.VMEM((D_O, _BPW), jnp.float32),
            pltpu.SemaphoreType.DMA,
        ],
    )(_sc_body)
    return k(idxv, idxj, wvt, wjt)


def kernel(TRA_v_gene, TRA_j_gene, W_v, W_j):
    zt = _gene_encode(
        TRA_v_gene.astype(jnp.int32),
        TRA_j_gene.astype(jnp.int32),
        W_v.T,
        W_j.T,
    )
    return zt.T


# speedup vs baseline: 5.9207x; 1.0063x over previous
import functools

import jax
import jax.numpy as jnp
from jax import lax
from jax.experimental import pallas as pl
from jax.experimental.pallas import tpu as pltpu
from jax.experimental.pallas import tpu_sc as plsc

B = 16384
V = 1000
D_V = 16
D_J = 8
D_O = D_V + D_J

_INFO = plsc.get_sparse_core_info()
_NC, _NS, _L = _INFO.num_cores, _INFO.num_subcores, _INFO.num_lanes
_NW = _NC * _NS
_BPW = B // _NW
_GRP = _BPW // _L


def _sc_body(idxv_hbm, idxj_hbm, wvt_hbm, wjt_hbm, out_hbm,
             idxv_v, idxj_v, wv_v, wj_v, comb_v, sem):
    wid = lax.axis_index("s") * _NC + lax.axis_index("c")
    base = wid * _BPW

    cps = [pltpu.async_copy(idxv_hbm.at[pl.ds(base, _BPW)], idxv_v, sem),
           pltpu.async_copy(idxj_hbm.at[pl.ds(base, _BPW)], idxj_v, sem),
           pltpu.async_copy(wvt_hbm, wv_v, sem),
           pltpu.async_copy(wjt_hbm, wj_v, sem)]
    for cp in cps:
        cp.wait()

    cvecs = [jnp.broadcast_to(jnp.int32(c), (_L,)) for c in range(D_V)]

    @plsc.parallel_loop(0, _GRP, unroll=2)
    def _group(g):
        off = g * _L
        ivv = idxv_v[pl.ds(off, _L)]
        ivj = idxj_v[pl.ds(off, _L)]
        for c in range(D_V):
            comb_v[c, pl.ds(off, _L)] = plsc.load_gather(wv_v, [cvecs[c], ivv])
        for c in range(D_J):
            comb_v[D_V + c, pl.ds(off, _L)] = plsc.load_gather(
                wj_v, [cvecs[c], ivj])

    pltpu.sync_copy(comb_v, out_hbm.at[:, pl.ds(base, _BPW)])


@jax.jit
def _gene_encode(idxv, idxj, wvt, wjt):
    mesh = plsc.VectorSubcoreMesh(core_axis_name="c", subcore_axis_name="s")
    k = functools.partial(
        pl.kernel,
        mesh=mesh,
        compiler_params=pltpu.CompilerParams(needs_layout_passes=False),
        out_type=jax.ShapeDtypeStruct((D_O, B), jnp.float32),
        scratch_types=[
            pltpu.VMEM((_BPW,), jnp.int32),
            pltpu.VMEM((_BPW,), jnp.int32),
            pltpu.VMEM((D_V, V), jnp.float32),
            pltpu.VMEM((D_J, V), jnp.float32),
            pltpu.VMEM((D_O, _BPW), jnp.float32),
            pltpu.SemaphoreType.DMA,
        ],
    )(_sc_body)
    return k(idxv, idxj, wvt, wjt)


def kernel(TRA_v_gene, TRA_j_gene, W_v, W_j):
    zt = _gene_encode(
        TRA_v_gene.astype(jnp.int32),
        TRA_j_gene.astype(jnp.int32),
        W_v.T,
        W_j.T,
    )
    return zt.T
